# baseline (speedup 1.0000x reference)
import functools

import jax
import jax.numpy as jnp
from jax.experimental import pallas as pl
from jax.experimental.pallas import tpu as pltpu

_BH = 32


def _gridpool_kernel(nblk, nr, row_id_ref, rh_ref, x_ref, wcol_ref, wcolt_ref,
                     out_ref, acc_ref, m_ref):
    g = pl.program_id(0)

    @pl.when(g == 0)
    def _zero():
        acc_ref[...] = jnp.zeros_like(acc_ref)

    @pl.when(g < nblk)
    def _reduce_rows():
        for l in range(_BH):
            rid = row_id_ref[g * _BH + l]
            acc_ref[pl.ds(rid, 1)] += x_ref[pl.ds(l, 1)]

    @pl.when(g == nblk)
    def _pool_scale_expand():
        for r in range(nr):
            a = acc_ref[r]
            m = jnp.dot(wcol_ref[...], a, preferred_element_type=jnp.float32)
            m_ref[r] = m * rh_ref[0, r]
        for r in range(nr):
            acc_ref[r] = jnp.dot(wcolt_ref[...], m_ref[r],
                                 preferred_element_type=jnp.float32)

    @pl.when(g >= nblk)
    def _emit():
        base = (g - nblk) * _BH
        for l in range(_BH):
            rid = row_id_ref[base + l]
            out_ref[pl.ds(l, 1)] = acc_ref[pl.ds(rid, 1)]


def kernel(input, h_positions, v_positions):
    x = input[0]
    h, w, c = x.shape
    p = h_positions.shape[0]
    q = v_positions.shape[0]
    nr, nq = p + 1, q + 1
    hp = h_positions.astype(jnp.int32)
    vp = v_positions.astype(jnp.int32)

    hb = jnp.concatenate([jnp.zeros((1,), jnp.int32), hp,
                          jnp.array([h], jnp.int32)])
    vb = jnp.concatenate([jnp.zeros((1,), jnp.int32), vp,
                          jnp.array([w], jnp.int32)])
    h_sizes = (hb[1:] - hb[:-1]).astype(jnp.float32)
    v_sizes = (vb[1:] - vb[:-1]).astype(jnp.float32)
    row_id = jnp.searchsorted(hp, jnp.arange(h, dtype=jnp.int32),
                              side='right').astype(jnp.int32)
    col_id = jnp.searchsorted(vp, jnp.arange(w, dtype=jnp.int32),
                              side='right').astype(jnp.int32)
    rh = jnp.where(h_sizes > 0,
                   1.0 / jnp.where(h_sizes > 0, h_sizes, 1.0), 0.0)
    rv = jnp.where(v_sizes > 0,
                   1.0 / jnp.where(v_sizes > 0, v_sizes, 1.0), 0.0)
    onehot_col = (col_id[None, :] ==
                  jnp.arange(nq, dtype=jnp.int32)[:, None]).astype(jnp.float32)
    w_col = onehot_col * rv[:, None]
    w_colt = onehot_col.T
    rh2 = rh[None, :]

    nblk = h // _BH
    body = functools.partial(_gridpool_kernel, nblk, nr)
    out = pl.pallas_call(
        body,
        grid=(2 * nblk,),
        in_specs=[
            pl.BlockSpec(memory_space=pltpu.SMEM),
            pl.BlockSpec(memory_space=pltpu.SMEM),
            pl.BlockSpec((_BH, w, c), lambda g: (jnp.minimum(g, nblk - 1), 0, 0)),
            pl.BlockSpec((nq, w), lambda g: (0, 0)),
            pl.BlockSpec((w, nq), lambda g: (0, 0)),
        ],
        out_specs=pl.BlockSpec((_BH, w, c),
                               lambda g: (jnp.maximum(g - nblk, 0), 0, 0)),
        out_shape=jax.ShapeDtypeStruct((h, w, c), jnp.float32),
        scratch_shapes=[
            pltpu.VMEM((nr, w, c), jnp.float32),
            pltpu.VMEM((nr, nq, c), jnp.float32),
        ],
    )(row_id, rh2, x, w_col, w_colt)
    return out[None]
